# baseline (speedup 1.0000x reference)
import functools

import jax
import jax.numpy as jnp
from jax import lax
from jax.experimental import pallas as pl
from jax.experimental.pallas import tpu as pltpu
from jax.experimental.pallas import tpu_sc as plsc

VOCAB = 100000
EMBED_DIM = 100
EMBED_PAD = 128
HIDDEN_DIM = 1024
OUT_DIM = 4
BATCH = 4096
HIST = 50

NUM_CORES = 2
NUM_SUBCORES = 16
NTILES = NUM_CORES * NUM_SUBCORES
SAMP_PER_SC = BATCH // NUM_CORES
SAMP_PER_TILE = BATCH // NTILES
CHUNK = 64
NCHUNK = SAMP_PER_TILE * HIST // CHUNK
NH = SAMP_PER_TILE // CHUNK


def _sc_pool(emb, idx3, oidx3, zrows):
    mesh = plsc.VectorSubcoreMesh(core_axis_name="c", subcore_axis_name="s")

    nbuf = 10

    @functools.partial(
        pl.kernel,
        mesh=mesh,
        out_type=jax.ShapeDtypeStruct((BATCH, EMBED_PAD), jnp.float32),
        scratch_types=(
            [pltpu.VMEM((NCHUNK, CHUNK), jnp.int32)]
            + [pltpu.VMEM((NH, CHUNK), jnp.int32)]
            + [pltpu.VMEM((CHUNK, EMBED_PAD), jnp.float32)] * nbuf
            + [pltpu.SemaphoreType.DMA] * (2 * nbuf)
            + [pltpu.VMEM_SHARED((SAMP_PER_SC, EMBED_PAD), jnp.float32)]
        ),
    )
    def k(emb_hbm, idx_hbm, oidx_hbm, z_hbm, out_hbm,
          idx_v, oidx_v, *rest):
        rows = list(rest[:nbuf])
        gsem = list(rest[nbuf:2 * nbuf])
        ssem = list(rest[2 * nbuf:3 * nbuf])
        pooled_sh = rest[3 * nbuf]
        c = lax.axis_index("c")
        s = lax.axis_index("s")
        t = c * NUM_SUBCORES + s
        pltpu.sync_copy(idx_hbm.at[t], idx_v)
        pltpu.sync_copy(oidx_hbm.at[t], oidx_v)
        pltpu.sync_copy(z_hbm.at[pl.ds(t * SAMP_PER_TILE, SAMP_PER_TILE)],
                        pooled_sh.at[pl.ds(s * SAMP_PER_TILE, SAMP_PER_TILE)])

        for b in range(nbuf):
            pltpu.async_copy(emb_hbm.at[idx_v.at[b]], rows[b], gsem[b])

        @pl.loop(0, (NCHUNK - nbuf) // nbuf)
        def _(g):
            j0 = g * nbuf
            for b in range(nbuf):
                j = j0 + b
                pltpu.make_async_copy(emb_hbm.at[idx_v.at[j]], rows[b], gsem[b]).wait()
                if b >= NH:
                    pltpu.make_async_copy(rows[b - NH],
                                          pooled_sh.at[oidx_v.at[b % NH]],
                                          ssem[b - NH]).wait()
                pltpu.async_copy(rows[b], pooled_sh.at[oidx_v.at[b % NH]],
                                 ssem[b], add=True)

            for b in range(nbuf):
                j = j0 + b
                if b >= nbuf - NH:
                    pltpu.make_async_copy(rows[b], pooled_sh.at[oidx_v.at[b % NH]],
                                          ssem[b]).wait()
                pltpu.async_copy(emb_hbm.at[idx_v.at[j + nbuf]], rows[b], gsem[b])

        for b in range(nbuf):
            j = NCHUNK - nbuf + b
            pltpu.make_async_copy(emb_hbm.at[idx_v.at[j]], rows[b], gsem[b]).wait()
            if b >= NH:
                pltpu.make_async_copy(rows[b - NH],
                                      pooled_sh.at[oidx_v.at[b % NH]],
                                      ssem[b - NH]).wait()
            pltpu.async_copy(rows[b], pooled_sh.at[oidx_v.at[b % NH]],
                             ssem[b], add=True)
        for b in range(nbuf - NH, nbuf):
            pltpu.make_async_copy(rows[b], pooled_sh.at[oidx_v.at[b % NH]],
                                  ssem[b]).wait()

        pltpu.sync_copy(pooled_sh.at[pl.ds(s * SAMP_PER_TILE, SAMP_PER_TILE)],
                        out_hbm.at[pl.ds(t * SAMP_PER_TILE, SAMP_PER_TILE)])

    return k(emb, idx3, oidx3, zrows)


_PAD_ROWS = 10000


def _tc_pad(emb):
    def body(x_ref, o_ref):
        o_ref[...] = jnp.pad(x_ref[...], ((0, 0), (0, EMBED_PAD - EMBED_DIM)))

    return pl.pallas_call(
        body,
        grid=(VOCAB // _PAD_ROWS,),
        in_specs=[pl.BlockSpec((_PAD_ROWS, EMBED_DIM), lambda i: (i, 0))],
        out_specs=pl.BlockSpec((_PAD_ROWS, EMBED_PAD), lambda i: (i, 0)),
        out_shape=jax.ShapeDtypeStruct((VOCAB, EMBED_PAD), jnp.float32),
    )(emb)


_BB = 512


def _tc_mlp(pooled, W1, b1, W2, b2):
    def body(p_ref, w1_ref, b1_ref, w2_ref, b2_ref, o_ref):
        h = jnp.dot(p_ref[...], w1_ref[...],
                    preferred_element_type=jnp.float32)
        h = jax.nn.sigmoid(h + b1_ref[...])
        o = jnp.dot(h, w2_ref[...],
                    preferred_element_type=jnp.float32)
        o_ref[...] = o + b2_ref[...]

    return pl.pallas_call(
        body,
        grid=(BATCH // _BB,),
        in_specs=[
            pl.BlockSpec((_BB, EMBED_PAD), lambda i: (i, 0)),
            pl.BlockSpec((EMBED_PAD, HIDDEN_DIM), lambda i: (0, 0)),
            pl.BlockSpec((1, HIDDEN_DIM), lambda i: (0, 0)),
            pl.BlockSpec((HIDDEN_DIM, OUT_DIM), lambda i: (0, 0)),
            pl.BlockSpec((1, OUT_DIM), lambda i: (0, 0)),
        ],
        out_specs=pl.BlockSpec((_BB, OUT_DIM), lambda i: (i, 0)),
        out_shape=jax.ShapeDtypeStruct((BATCH, OUT_DIM), jnp.float32),
    )(pooled, W1, b1.reshape(1, HIDDEN_DIM), W2, b2.reshape(1, OUT_DIM))


def kernel(x, emb, W1, b1, W2, b2):
    idx3 = (
        x.reshape(NTILES, NH, CHUNK, HIST)
        .transpose(0, 3, 1, 2)
        .reshape(NTILES, NCHUNK, CHUNK)
    )
    oidx3 = (
        (jnp.arange(NTILES, dtype=jnp.int32)[:, None] % NUM_SUBCORES)
        * SAMP_PER_TILE
        + jnp.arange(SAMP_PER_TILE, dtype=jnp.int32)[None, :]
    ).reshape(NTILES, NH, CHUNK)
    zrows = jnp.zeros((BATCH, EMBED_PAD), jnp.float32)
    W1p = jnp.pad(W1, ((0, EMBED_PAD - EMBED_DIM), (0, 0)))
    embp = _tc_pad(emb)
    pooled = _sc_pool(embp, idx3, oidx3, zrows)
    return _tc_mlp(pooled, W1p, b1, W2, b2)


# speedup vs baseline: 1.0875x; 1.0875x over previous
import functools

import jax
import jax.numpy as jnp
from jax import lax
from jax.experimental import pallas as pl
from jax.experimental.pallas import tpu as pltpu
from jax.experimental.pallas import tpu_sc as plsc

VOCAB = 100000
EMBED_DIM = 100
EMBED_PAD = 128
HIDDEN_DIM = 1024
OUT_DIM = 4
BATCH = 4096
HIST = 50

NUM_CORES = 2
NUM_SUBCORES = 16
NTILES = NUM_CORES * NUM_SUBCORES
SAMP_PER_SC = BATCH // NUM_CORES
SAMP_PER_TILE = BATCH // NTILES
CHUNK = 64
NCHUNK = SAMP_PER_TILE * HIST // CHUNK
NH = SAMP_PER_TILE // CHUNK


def _sc_pool(emb, idx3, oidx3, zrows):
    mesh = plsc.VectorSubcoreMesh(core_axis_name="c", subcore_axis_name="s")

    nbuf = 10

    @functools.partial(
        pl.kernel,
        mesh=mesh,
        out_type=jax.ShapeDtypeStruct((BATCH, EMBED_PAD), jnp.float32),
        scratch_types=(
            [pltpu.VMEM((NCHUNK, CHUNK), jnp.int32)]
            + [pltpu.VMEM((NH, CHUNK), jnp.int32)]
            + [pltpu.VMEM((CHUNK, EMBED_PAD), jnp.float32)] * nbuf
            + [pltpu.SemaphoreType.DMA] * (2 * nbuf)
            + [pltpu.VMEM_SHARED((SAMP_PER_SC, EMBED_PAD), jnp.float32)]
        ),
    )
    def k(emb_hbm, idx_hbm, oidx_hbm, z_hbm, out_hbm,
          idx_v, oidx_v, *rest):
        rows = list(rest[:nbuf])
        gsem = list(rest[nbuf:2 * nbuf])
        ssem = list(rest[2 * nbuf:3 * nbuf])
        pooled_sh = rest[3 * nbuf]
        c = lax.axis_index("c")
        s = lax.axis_index("s")
        t = c * NUM_SUBCORES + s
        pltpu.sync_copy(idx_hbm.at[t], idx_v)
        pltpu.sync_copy(oidx_hbm.at[t], oidx_v)
        pltpu.sync_copy(z_hbm.at[pl.ds(t * SAMP_PER_TILE, SAMP_PER_TILE)],
                        pooled_sh.at[pl.ds(s * SAMP_PER_TILE, SAMP_PER_TILE)])

        for b in range(nbuf):
            pltpu.async_copy(emb_hbm.at[idx_v.at[b]], rows[b], gsem[b])

        @pl.loop(0, (NCHUNK - nbuf) // nbuf)
        def _(g):
            j0 = g * nbuf
            for b in range(nbuf):
                j = j0 + b
                pltpu.make_async_copy(emb_hbm.at[idx_v.at[j]], rows[b], gsem[b]).wait()
                if b >= NH:
                    pltpu.make_async_copy(rows[b - NH],
                                          pooled_sh.at[oidx_v.at[b % NH]],
                                          ssem[b - NH]).wait()
                    pltpu.async_copy(emb_hbm.at[idx_v.at[j0 + b - NH + nbuf]],
                                     rows[b - NH], gsem[b - NH])
                pltpu.async_copy(rows[b], pooled_sh.at[oidx_v.at[b % NH]],
                                 ssem[b], add=True)

            for b in range(nbuf - NH, nbuf):
                pltpu.make_async_copy(rows[b], pooled_sh.at[oidx_v.at[b % NH]],
                                      ssem[b]).wait()
                pltpu.async_copy(emb_hbm.at[idx_v.at[j0 + b + nbuf]], rows[b], gsem[b])

        for b in range(nbuf):
            j = NCHUNK - nbuf + b
            pltpu.make_async_copy(emb_hbm.at[idx_v.at[j]], rows[b], gsem[b]).wait()
            if b >= NH:
                pltpu.make_async_copy(rows[b - NH],
                                      pooled_sh.at[oidx_v.at[b % NH]],
                                      ssem[b - NH]).wait()
            pltpu.async_copy(rows[b], pooled_sh.at[oidx_v.at[b % NH]],
                             ssem[b], add=True)
        for b in range(nbuf - NH, nbuf):
            pltpu.make_async_copy(rows[b], pooled_sh.at[oidx_v.at[b % NH]],
                                  ssem[b]).wait()

        pltpu.sync_copy(pooled_sh.at[pl.ds(s * SAMP_PER_TILE, SAMP_PER_TILE)],
                        out_hbm.at[pl.ds(t * SAMP_PER_TILE, SAMP_PER_TILE)])

    return k(emb, idx3, oidx3, zrows)


_PAD_ROWS = 10000


def _tc_pad(emb):
    def body(x_ref, o_ref):
        o_ref[...] = jnp.pad(x_ref[...], ((0, 0), (0, EMBED_PAD - EMBED_DIM)))

    return pl.pallas_call(
        body,
        grid=(VOCAB // _PAD_ROWS,),
        in_specs=[pl.BlockSpec((_PAD_ROWS, EMBED_DIM), lambda i: (i, 0))],
        out_specs=pl.BlockSpec((_PAD_ROWS, EMBED_PAD), lambda i: (i, 0)),
        out_shape=jax.ShapeDtypeStruct((VOCAB, EMBED_PAD), jnp.float32),
    )(emb)


_BB = 512


def _tc_mlp(pooled, W1, b1, W2, b2):
    def body(p_ref, w1_ref, b1_ref, w2_ref, b2_ref, o_ref):
        h = jnp.dot(p_ref[...], w1_ref[...],
                    preferred_element_type=jnp.float32)
        h = jax.nn.sigmoid(h + b1_ref[...])
        o = jnp.dot(h, w2_ref[...],
                    preferred_element_type=jnp.float32)
        o_ref[...] = o + b2_ref[...]

    return pl.pallas_call(
        body,
        grid=(BATCH // _BB,),
        in_specs=[
            pl.BlockSpec((_BB, EMBED_PAD), lambda i: (i, 0)),
            pl.BlockSpec((EMBED_PAD, HIDDEN_DIM), lambda i: (0, 0)),
            pl.BlockSpec((1, HIDDEN_DIM), lambda i: (0, 0)),
            pl.BlockSpec((HIDDEN_DIM, OUT_DIM), lambda i: (0, 0)),
            pl.BlockSpec((1, OUT_DIM), lambda i: (0, 0)),
        ],
        out_specs=pl.BlockSpec((_BB, OUT_DIM), lambda i: (i, 0)),
        out_shape=jax.ShapeDtypeStruct((BATCH, OUT_DIM), jnp.float32),
    )(pooled, W1, b1.reshape(1, HIDDEN_DIM), W2, b2.reshape(1, OUT_DIM))


def kernel(x, emb, W1, b1, W2, b2):
    idx3 = (
        x.reshape(NTILES, NH, CHUNK, HIST)
        .transpose(0, 3, 1, 2)
        .reshape(NTILES, NCHUNK, CHUNK)
    )
    oidx3 = (
        (jnp.arange(NTILES, dtype=jnp.int32)[:, None] % NUM_SUBCORES)
        * SAMP_PER_TILE
        + jnp.arange(SAMP_PER_TILE, dtype=jnp.int32)[None, :]
    ).reshape(NTILES, NH, CHUNK)
    zrows = jnp.zeros((BATCH, EMBED_PAD), jnp.float32)
    W1p = jnp.pad(W1, ((0, EMBED_PAD - EMBED_DIM), (0, 0)))
    embp = _tc_pad(emb)
    pooled = _sc_pool(embp, idx3, oidx3, zrows)
    return _tc_mlp(pooled, W1p, b1, W2, b2)
